# Initial kernel scaffold; baseline (speedup 1.0000x reference)
import jax
import jax.numpy as jnp
from jax.experimental import pallas as pl


def kernel(src_idx_l, target_idx_l, edge_idxs, cut_time_l, node_raw_features, edge_raw_features, ngh_node_table, ngh_eidx_table, ngh_ts_table, time_basis_freq, time_phase, Wq, Wk, Wv, Wo, merge_W1, merge_b1, merge_W2, merge_b2, aff_W1, aff_b1, aff_W2, aff_b2):
    raise NotImplementedError("write your pallas kernel here")


# speedup vs baseline: 2.7125x; 2.7125x over previous
import functools

import jax
import jax.numpy as jnp
import numpy as np
from jax.experimental import pallas as pl

N_NODES = 10000
N_EDGES = 320000
FEAT_DIM = 128
TIME_DIM = 128
N_HEAD = 4
K_NGH = 20
Q_DIM = FEAT_DIM + TIME_DIM
K_DIM = FEAT_DIM + FEAT_DIM + TIME_DIM
D_HEAD = Q_DIM // N_HEAD

_E_NP = np.zeros((Q_DIM, N_HEAD), dtype=np.float32)
for _h in range(N_HEAD):
    _E_NP[_h * D_HEAD:(_h + 1) * D_HEAD, _h] = 1.0


def _attn_body(R, sc_ref, sf_ref, ngh_ref, edge_ref, ts_ref, t_ref, nid_ref,
               wq_ref, wkv_ref, wo_ref, w1_ref, b1_ref, w2_ref, b2_ref,
               freq_ref, phase_ref, e_ref, et_ref, out_ref):
    F = R * K_NGH
    delta = t_ref[...] - ts_ref[...]
    tenc = jnp.cos(delta * freq_ref[...] + phase_ref[...])
    kmat = jnp.concatenate([ngh_ref[...], edge_ref[...], tenc], axis=1)
    kv = jnp.dot(kmat, wkv_ref[...], preferred_element_type=jnp.float32)
    k = kv[:, :Q_DIM]
    v = kv[:, Q_DIM:]
    srct = jnp.cos(phase_ref[...])
    qb = jnp.dot(srct, wq_ref[...][FEAT_DIM:, :], preferred_element_type=jnp.float32)
    qh = jnp.dot(sc_ref[...], wq_ref[...][:FEAT_DIM, :], preferred_element_type=jnp.float32) + qb
    qexp = jnp.broadcast_to(qh.reshape(R, 1, Q_DIM), (R, K_NGH, Q_DIM)).reshape(F, Q_DIM)
    p = k * qexp
    scores = jnp.dot(p, e_ref[...], preferred_element_type=jnp.float32) * (1.0 / 8.0)
    scores = jnp.where(nid_ref[...] == 0, jnp.float32(-1e10), scores)
    s3 = scores.reshape(R, K_NGH, N_HEAD)
    m = jnp.max(s3, axis=1, keepdims=True)
    ex = jnp.exp(s3 - m)
    ssum = jnp.sum(ex, axis=1, keepdims=True)
    attn = (ex / ssum).reshape(F, N_HEAD)
    aexp = jnp.dot(attn, et_ref[...], preferred_element_type=jnp.float32)
    ctx = (aexp * v).reshape(R, K_NGH, Q_DIM).sum(axis=1)
    local = jnp.dot(ctx, wo_ref[...], preferred_element_type=jnp.float32)
    h = jnp.concatenate([local, sf_ref[...]], axis=1)
    h = jnp.dot(h, w1_ref[...], preferred_element_type=jnp.float32) + b1_ref[...]
    h = jnp.maximum(h, 0.0)
    out_ref[...] = jnp.dot(h, w2_ref[...], preferred_element_type=jnp.float32) + b2_ref[...]


def _attn_layer(srcconv, srcfeat, nghfeat, edgefeat, ts_f, t_f, nid_f,
                Wq, Wkv, Wo, mW1, mb1, mW2, mb2, freq, phase, R):
    B = srcconv.shape[0]
    F = R * K_NGH
    grid = (B // R,)
    row_spec = pl.BlockSpec((R, FEAT_DIM), lambda i: (i, 0))
    flat_spec = pl.BlockSpec((F, FEAT_DIM), lambda i: (i, 0))
    col_spec = pl.BlockSpec((F, 1), lambda i: (i, 0))
    full = lambda a: pl.BlockSpec(a.shape, lambda i: tuple(0 for _ in a.shape))
    e_mat = jnp.asarray(_E_NP)
    et_mat = jnp.asarray(_E_NP.T)
    consts = (Wq, Wkv, Wo, mW1, mb1, mW2, mb2, freq, phase, e_mat, et_mat)
    return pl.pallas_call(
        functools.partial(_attn_body, R),
        grid=grid,
        in_specs=[row_spec, row_spec, flat_spec, flat_spec, col_spec, col_spec,
                  col_spec] + [full(c) for c in consts],
        out_specs=row_spec,
        out_shape=jax.ShapeDtypeStruct((B, FEAT_DIM), jnp.float32),
    )(srcconv, srcfeat, nghfeat, edgefeat, ts_f, t_f, nid_f, *consts)


def kernel(src_idx_l, target_idx_l, edge_idxs, cut_time_l, node_raw_features,
           edge_raw_features, ngh_node_table, ngh_eidx_table, ngh_ts_table,
           time_basis_freq, time_phase, Wq, Wk, Wv, Wo, merge_W1, merge_b1,
           merge_W2, merge_b2, aff_W1, aff_b1, aff_W2, aff_b2):
    B2 = src_idx_l.shape[0]
    freq = time_basis_freq.reshape(1, TIME_DIM)
    phase = time_phase.reshape(1, TIME_DIM)

    ngh2 = jnp.take(ngh_node_table, src_idx_l, axis=0)
    eidx2 = jnp.take(ngh_eidx_table, src_idx_l, axis=0)
    ts2 = jnp.take(ngh_ts_table, src_idx_l, axis=0)

    idx1 = jnp.concatenate([src_idx_l, ngh2.reshape(-1)])
    t1 = jnp.concatenate([cut_time_l, ts2.reshape(-1)])
    B1 = idx1.shape[0]

    ngh1 = jnp.take(ngh_node_table, idx1, axis=0)
    eidx1 = jnp.take(ngh_eidx_table, idx1, axis=0)
    ts1 = jnp.take(ngh_ts_table, idx1, axis=0)

    srcfeat1 = jnp.take(node_raw_features, idx1, axis=0)
    nghfeat1 = jnp.take(node_raw_features, ngh1.reshape(-1), axis=0)
    edgefeat1 = jnp.take(edge_raw_features, eidx1.reshape(-1), axis=0)
    edgefeat2 = jnp.take(edge_raw_features, eidx2.reshape(-1), axis=0)

    Wkv0 = jnp.concatenate([Wk[0], Wv[0]], axis=1)
    Wkv1 = jnp.concatenate([Wk[1], Wv[1]], axis=1)

    ts1_f = ts1.reshape(-1, 1)
    t1_f = jnp.repeat(t1, K_NGH).reshape(-1, 1)
    nid1_f = ngh1.reshape(-1, 1)

    out1 = _attn_layer(srcfeat1, srcfeat1, nghfeat1, edgefeat1, ts1_f, t1_f,
                       nid1_f, Wq[0], Wkv0, Wo[0], merge_W1[0],
                       merge_b1[0].reshape(1, -1), merge_W2[0],
                       merge_b2[0].reshape(1, -1), freq, phase, R=128)

    src_conv2 = out1[:B2]
    ngh_conv2 = out1[B2:]

    ts2_f = ts2.reshape(-1, 1)
    t2_f = jnp.repeat(cut_time_l, K_NGH).reshape(-1, 1)
    nid2_f = ngh2.reshape(-1, 1)

    out2 = _attn_layer(src_conv2, srcfeat1[:B2], ngh_conv2, edgefeat2, ts2_f,
                       t2_f, nid2_f, Wq[1], Wkv1, Wo[1], merge_W1[1],
                       merge_b1[1].reshape(1, -1), merge_W2[1],
                       merge_b2[1].reshape(1, -1), freq, phase, R=128)
    return out2


# speedup vs baseline: 5.1568x; 1.9011x over previous
import functools

import jax
import jax.numpy as jnp
import numpy as np
from jax import lax
from jax.experimental import pallas as pl
from jax.experimental.pallas import tpu as pltpu
from jax.experimental.pallas import tpu_sc as plsc

N_NODES = 10000
N_EDGES = 320000
FEAT_DIM = 128
TIME_DIM = 128
N_HEAD = 4
K_NGH = 20
Q_DIM = FEAT_DIM + TIME_DIM
K_DIM = FEAT_DIM + FEAT_DIM + TIME_DIM
D_HEAD = Q_DIM // N_HEAD

_E_NP = np.zeros((Q_DIM, N_HEAD), dtype=np.float32)
for _h in range(N_HEAD):
    _E_NP[_h * D_HEAD:(_h + 1) * D_HEAD, _h] = 1.0


def _attn_body(R, sc_ref, sf_ref, ngh_ref, edge_ref, ts_ref, t_ref, nid_ref,
               wq_ref, wkv_ref, wo_ref, w1_ref, b1_ref, w2_ref, b2_ref,
               freq_ref, phase_ref, e_ref, et_ref, out_ref):
    F = R * K_NGH
    delta = t_ref[...] - ts_ref[...]
    tenc = jnp.cos(delta * freq_ref[...] + phase_ref[...])
    kmat = jnp.concatenate([ngh_ref[...], edge_ref[...], tenc], axis=1)
    kv = jnp.dot(kmat, wkv_ref[...], preferred_element_type=jnp.float32)
    k = kv[:, :Q_DIM]
    v = kv[:, Q_DIM:]
    srct = jnp.cos(phase_ref[...])
    qb = jnp.dot(srct, wq_ref[...][FEAT_DIM:, :], preferred_element_type=jnp.float32)
    qh = jnp.dot(sc_ref[...], wq_ref[...][:FEAT_DIM, :], preferred_element_type=jnp.float32) + qb
    qexp = jnp.broadcast_to(qh.reshape(R, 1, Q_DIM), (R, K_NGH, Q_DIM)).reshape(F, Q_DIM)
    p = k * qexp
    scores = jnp.dot(p, e_ref[...], preferred_element_type=jnp.float32) * (1.0 / 8.0)
    scores = jnp.where(nid_ref[...] == 0, jnp.float32(-1e10), scores)
    s3 = scores.reshape(R, K_NGH, N_HEAD)
    m = jnp.max(s3, axis=1, keepdims=True)
    ex = jnp.exp(s3 - m)
    ssum = jnp.sum(ex, axis=1, keepdims=True)
    attn = (ex / ssum).reshape(F, N_HEAD)
    aexp = jnp.dot(attn, et_ref[...], preferred_element_type=jnp.float32)
    ctx = (aexp * v).reshape(R, K_NGH, Q_DIM).sum(axis=1)
    local = jnp.dot(ctx, wo_ref[...], preferred_element_type=jnp.float32)
    h = jnp.concatenate([local, sf_ref[...]], axis=1)
    h = jnp.dot(h, w1_ref[...], preferred_element_type=jnp.float32) + b1_ref[...]
    h = jnp.maximum(h, 0.0)
    out_ref[...] = jnp.dot(h, w2_ref[...], preferred_element_type=jnp.float32) + b2_ref[...]


def _attn_layer(srcconv, srcfeat, nghfeat, edgefeat, ts_f, t_f, nid_f,
                Wq, Wkv, Wo, mW1, mb1, mW2, mb2, freq, phase, R):
    B = srcconv.shape[0]
    F = R * K_NGH
    grid = (B // R,)
    row_spec = pl.BlockSpec((R, FEAT_DIM), lambda i: (i, 0))
    flat_spec = pl.BlockSpec((F, FEAT_DIM), lambda i: (i, 0))
    col_spec = pl.BlockSpec((F, 1), lambda i: (i, 0))
    full = lambda a: pl.BlockSpec(a.shape, lambda i: tuple(0 for _ in a.shape))
    e_mat = jnp.asarray(_E_NP)
    et_mat = jnp.asarray(_E_NP.T)
    consts = (Wq, Wkv, Wo, mW1, mb1, mW2, mb2, freq, phase, e_mat, et_mat)
    return pl.pallas_call(
        functools.partial(_attn_body, R),
        grid=grid,
        in_specs=[row_spec, row_spec, flat_spec, flat_spec, col_spec, col_spec,
                  col_spec] + [full(c) for c in consts],
        out_specs=row_spec,
        out_shape=jax.ShapeDtypeStruct((B, FEAT_DIM), jnp.float32),
    )(srcconv, srcfeat, nghfeat, edgefeat, ts_f, t_f, nid_f, *consts)


_NW = 32


def _pick_chunk(bpw):
    for c in range(min(bpw, 128), 7, -1):
        if bpw % c == 0 and c % 8 == 0:
            return c
    return bpw


def _sc_gather(table, idx):
    B = idx.shape[0]
    D = table.shape[1]
    bpw = B // _NW
    C = _pick_chunk(bpw)
    nch = bpw // C
    mesh = plsc.VectorSubcoreMesh(core_axis_name="c", subcore_axis_name="s")

    @functools.partial(
        pl.kernel, mesh=mesh,
        out_type=jax.ShapeDtypeStruct((B, D), table.dtype),
        scratch_types=[
            pltpu.VMEM((bpw,), jnp.int32),
            pltpu.VMEM((2, C, D), table.dtype),
            pltpu.SemaphoreType.DMA,
        ],
    )
    def gk(table_hbm, idx_hbm, out_hbm, idx_v, buf_v, gsem):
        wid = lax.axis_index("s") * 2 + lax.axis_index("c")
        base = wid * bpw
        pltpu.sync_copy(idx_hbm.at[pl.ds(base, bpw)], idx_v)
        cp = pltpu.async_copy(table_hbm.at[idx_v.at[pl.ds(0, C)]],
                              buf_v.at[0], gsem)

        def body(g, _):
            slot = lax.rem(g, 2)
            nxt = lax.rem(g + 1, 2)

            @pl.when(g + 1 < nch)
            def _():
                pltpu.async_copy(
                    table_hbm.at[idx_v.at[pl.ds((g + 1) * C, C)]],
                    buf_v.at[nxt], gsem)

            pltpu.make_async_copy(table_hbm.at[idx_v.at[pl.ds(0, C)]],
                                  buf_v.at[slot], gsem).wait()
            pltpu.sync_copy(buf_v.at[slot], out_hbm.at[pl.ds(base + g * C, C)])
            return 0

        lax.fori_loop(0, nch, body, 0)

    return gk(table, idx)


def kernel(src_idx_l, target_idx_l, edge_idxs, cut_time_l, node_raw_features,
           edge_raw_features, ngh_node_table, ngh_eidx_table, ngh_ts_table,
           time_basis_freq, time_phase, Wq, Wk, Wv, Wo, merge_W1, merge_b1,
           merge_W2, merge_b2, aff_W1, aff_b1, aff_W2, aff_b2):
    B2 = src_idx_l.shape[0]
    freq = time_basis_freq.reshape(1, TIME_DIM)
    phase = time_phase.reshape(1, TIME_DIM)

    ngh2 = jnp.take(ngh_node_table, src_idx_l, axis=0)
    eidx2 = jnp.take(ngh_eidx_table, src_idx_l, axis=0)
    ts2 = jnp.take(ngh_ts_table, src_idx_l, axis=0)

    idx1 = jnp.concatenate([src_idx_l, ngh2.reshape(-1)])
    t1 = jnp.concatenate([cut_time_l, ts2.reshape(-1)])
    B1 = idx1.shape[0]

    ngh1 = jnp.take(ngh_node_table, idx1, axis=0)
    eidx1 = jnp.take(ngh_eidx_table, idx1, axis=0)
    ts1 = jnp.take(ngh_ts_table, idx1, axis=0)

    srcfeat1 = _sc_gather(node_raw_features, idx1)
    nghfeat1 = _sc_gather(node_raw_features, ngh1.reshape(-1))
    edgefeat1 = _sc_gather(edge_raw_features, eidx1.reshape(-1))
    edgefeat2 = _sc_gather(edge_raw_features, eidx2.reshape(-1))

    Wkv0 = jnp.concatenate([Wk[0], Wv[0]], axis=1)
    Wkv1 = jnp.concatenate([Wk[1], Wv[1]], axis=1)

    ts1_f = ts1.reshape(-1, 1)
    t1_f = jnp.repeat(t1, K_NGH).reshape(-1, 1)
    nid1_f = ngh1.reshape(-1, 1)

    out1 = _attn_layer(srcfeat1, srcfeat1, nghfeat1, edgefeat1, ts1_f, t1_f,
                       nid1_f, Wq[0], Wkv0, Wo[0], merge_W1[0],
                       merge_b1[0].reshape(1, -1), merge_W2[0],
                       merge_b2[0].reshape(1, -1), freq, phase, R=128)

    src_conv2 = out1[:B2]
    ngh_conv2 = out1[B2:]

    ts2_f = ts2.reshape(-1, 1)
    t2_f = jnp.repeat(cut_time_l, K_NGH).reshape(-1, 1)
    nid2_f = ngh2.reshape(-1, 1)

    out2 = _attn_layer(src_conv2, srcfeat1[:B2], ngh_conv2, edgefeat2, ts2_f,
                       t2_f, nid2_f, Wq[1], Wkv1, Wo[1], merge_W1[1],
                       merge_b1[1].reshape(1, -1), merge_W2[1],
                       merge_b2[1].reshape(1, -1), freq, phase, R=128)
    return out2


# speedup vs baseline: 9.9294x; 1.9255x over previous
import functools

import jax
import jax.numpy as jnp
import numpy as np
from jax import lax
from jax.experimental import pallas as pl
from jax.experimental.pallas import tpu as pltpu
from jax.experimental.pallas import tpu_sc as plsc

N_NODES = 10000
N_EDGES = 320000
FEAT_DIM = 128
TIME_DIM = 128
N_HEAD = 4
K_NGH = 20
Q_DIM = FEAT_DIM + TIME_DIM
K_DIM = FEAT_DIM + FEAT_DIM + TIME_DIM
D_HEAD = Q_DIM // N_HEAD

_E_NP = np.zeros((Q_DIM, N_HEAD), dtype=np.float32)
for _h in range(N_HEAD):
    _E_NP[_h * D_HEAD:(_h + 1) * D_HEAD, _h] = 1.0

_RB = 128
_FB = _RB * K_NGH
_L_NP = np.zeros((_FB, 128), dtype=np.float32)
for _f in range(_FB):
    _L_NP[_f, _f % 128] = 1.0

_INV2PI = 0.15915494309189535
_COS_COEF = (0.9999999999190519, -19.739208758070983, 64.93939010304335,
             -85.45668509273975, 60.24246091859294, -26.40673585263105,
             7.8065259948731125, -1.4608429364038422)
def _fast_cos_rev(t):
    r = t - jnp.round(t)
    u = r * r
    acc = jnp.float32(_COS_COEF[-1])
    for c in _COS_COEF[-2::-1]:
        acc = acc * u + jnp.float32(c)
    return acc


def _attn_body(R, sc_ref, sf_ref, ngh_ref, edge_ref, d_ref, m_ref,
               wq_ref, wkv_ref, wo_ref, w1_ref, b1_ref, w2_ref, b2_ref,
               freq_ref, phase_ref, e_ref, et_ref, l_ref, out_ref):
    F = R * K_NGH
    lmat = l_ref[...]

    def expand(pk):
        y = jnp.broadcast_to(pk[:, None, :], (F // 128, 128, 128)).reshape(F, 128)
        return jnp.sum(y * lmat, axis=1, keepdims=True)

    delta = expand(d_ref[0])
    mask = expand(m_ref[0])
    tenc = _fast_cos_rev(delta * freq_ref[...] + phase_ref[...])
    kmat = jnp.concatenate([ngh_ref[...], edge_ref[...], tenc], axis=1)
    kv = jnp.dot(kmat, wkv_ref[...], preferred_element_type=jnp.float32)
    k = kv[:, :Q_DIM]
    v = kv[:, Q_DIM:]
    srct = _fast_cos_rev(phase_ref[...])
    qb = jnp.dot(srct, wq_ref[...][FEAT_DIM:, :], preferred_element_type=jnp.float32)
    qh = jnp.dot(sc_ref[...], wq_ref[...][:FEAT_DIM, :], preferred_element_type=jnp.float32) + qb
    qexp = jnp.broadcast_to(qh.reshape(R, 1, Q_DIM), (R, K_NGH, Q_DIM)).reshape(F, Q_DIM)
    p = k * qexp
    scores = jnp.dot(p, e_ref[...], preferred_element_type=jnp.float32) * (1.0 / 8.0)
    scores = jnp.where(mask > 0.5, jnp.float32(-1e10), scores)
    s3 = scores.reshape(R, K_NGH, N_HEAD)
    m = jnp.max(s3, axis=1, keepdims=True)
    ex = jnp.exp(s3 - m)
    ssum = jnp.sum(ex, axis=1, keepdims=True)
    attn = (ex / ssum).reshape(F, N_HEAD)
    aexp = jnp.dot(attn, et_ref[...], preferred_element_type=jnp.float32)
    ctx = (aexp * v).reshape(R, K_NGH, Q_DIM).sum(axis=1)
    local = jnp.dot(ctx, wo_ref[...], preferred_element_type=jnp.float32)
    h = jnp.concatenate([local, sf_ref[...]], axis=1)
    h = jnp.dot(h, w1_ref[...], preferred_element_type=jnp.float32) + b1_ref[...]
    h = jnp.maximum(h, 0.0)
    out_ref[...] = jnp.dot(h, w2_ref[...], preferred_element_type=jnp.float32) + b2_ref[...]


def _attn_layer(B, srcconv, srcfeat, nghfeat, edgefeat, d_pk, m_pk,
                Wq, Wkv, Wo, mW1, mb1, mW2, mb2, freq, phase, pk_off=0):
    R = _RB
    F = R * K_NGH
    grid = (B // R,)

    def row_spec(off):
        assert off % R == 0
        ob = off // R
        return pl.BlockSpec((R, FEAT_DIM), lambda i: (ob + i, 0))

    def flat_spec(off):
        assert off % F == 0
        ob = off // F
        return pl.BlockSpec((F, FEAT_DIM), lambda i: (ob + i, 0))

    pk_ob = pk_off
    pk_spec = pl.BlockSpec((1, F // 128, 128), lambda i: (pk_ob + i, 0, 0))
    full = lambda a: pl.BlockSpec(a.shape, lambda i: tuple(0 for _ in a.shape))
    consts = (Wq, Wkv, Wo, mW1, mb1, mW2, mb2, freq, phase,
              jnp.asarray(_E_NP), jnp.asarray(_E_NP.T), jnp.asarray(_L_NP))
    return pl.pallas_call(
        functools.partial(_attn_body, R),
        grid=grid,
        in_specs=[row_spec(srcconv[1]), row_spec(srcfeat[1]),
                  flat_spec(nghfeat[1]), flat_spec(edgefeat[1]),
                  pk_spec, pk_spec] + [full(c) for c in consts],
        out_specs=pl.BlockSpec((R, FEAT_DIM), lambda i: (i, 0)),
        out_shape=jax.ShapeDtypeStruct((B, FEAT_DIM), jnp.float32),
    )(srcconv[0], srcfeat[0], nghfeat[0], edgefeat[0], d_pk, m_pk, *consts)


_NW = 32


def _pick_chunk(bpw):
    for c in range(min(bpw, 128), 7, -1):
        if bpw % c == 0 and c % 8 == 0:
            return c
    return bpw


def _sc_gather(table, idx):
    B = idx.shape[0]
    D = table.shape[1]
    bpw = B // _NW
    C = _pick_chunk(bpw)
    nch = bpw // C
    mesh = plsc.VectorSubcoreMesh(core_axis_name="c", subcore_axis_name="s")

    @functools.partial(
        pl.kernel, mesh=mesh,
        out_type=jax.ShapeDtypeStruct((B, D), table.dtype),
        scratch_types=[
            pltpu.VMEM((bpw,), jnp.int32),
            pltpu.VMEM((2, C, D), table.dtype),
            pltpu.SemaphoreType.DMA,
        ],
    )
    def gk(table_hbm, idx_hbm, out_hbm, idx_v, buf_v, gsem):
        wid = lax.axis_index("s") * 2 + lax.axis_index("c")
        base = wid * bpw
        pltpu.sync_copy(idx_hbm.at[pl.ds(base, bpw)], idx_v)
        cp = pltpu.async_copy(table_hbm.at[idx_v.at[pl.ds(0, C)]],
                              buf_v.at[0], gsem)

        def body(g, _):
            slot = lax.rem(g, 2)
            nxt = lax.rem(g + 1, 2)

            @pl.when(g + 1 < nch)
            def _():
                pltpu.async_copy(
                    table_hbm.at[idx_v.at[pl.ds((g + 1) * C, C)]],
                    buf_v.at[nxt], gsem)

            pltpu.make_async_copy(table_hbm.at[idx_v.at[pl.ds(0, C)]],
                                  buf_v.at[slot], gsem).wait()
            pltpu.sync_copy(buf_v.at[slot], out_hbm.at[pl.ds(base + g * C, C)])
            return 0

        lax.fori_loop(0, nch, body, 0)

    return gk(table, idx)


def kernel(src_idx_l, target_idx_l, edge_idxs, cut_time_l, node_raw_features,
           edge_raw_features, ngh_node_table, ngh_eidx_table, ngh_ts_table,
           time_basis_freq, time_phase, Wq, Wk, Wv, Wo, merge_W1, merge_b1,
           merge_W2, merge_b2, aff_W1, aff_b1, aff_W2, aff_b2):
    B2 = src_idx_l.shape[0]
    freq = time_basis_freq.reshape(1, TIME_DIM) * _INV2PI
    phase = time_phase.reshape(1, TIME_DIM) * _INV2PI

    ctable = jnp.concatenate(
        [ngh_node_table, ngh_eidx_table,
         lax.bitcast_convert_type(ngh_ts_table, jnp.int32),
         jnp.zeros((ngh_node_table.shape[0], 68), jnp.int32)], axis=1)

    tbl2 = _sc_gather(ctable, src_idx_l)
    ngh2 = tbl2[:, :K_NGH]
    eidx2 = tbl2[:, K_NGH:2 * K_NGH]
    ts2 = lax.bitcast_convert_type(tbl2[:, 2 * K_NGH:3 * K_NGH], jnp.float32)

    idx1 = jnp.concatenate([ngh2.reshape(-1), src_idx_l])
    t1 = jnp.concatenate([ts2.reshape(-1), cut_time_l])
    B1 = idx1.shape[0]
    NGH2F = B2 * K_NGH

    tbl1 = _sc_gather(ctable, idx1)
    ngh1 = tbl1[:, :K_NGH]
    eidx1 = tbl1[:, K_NGH:2 * K_NGH]
    ts1 = lax.bitcast_convert_type(tbl1[:, 2 * K_NGH:3 * K_NGH], jnp.float32)

    H = B1 // 2
    HF = H * K_NGH
    nodefeatA = _sc_gather(node_raw_features,
                           jnp.concatenate([ngh1[:H].reshape(-1), idx1[:H]]))
    edgefeatA = _sc_gather(edge_raw_features, eidx1[:H].reshape(-1))
    nodefeatB = _sc_gather(node_raw_features,
                           jnp.concatenate([ngh1[H:].reshape(-1), idx1[H:]]))
    edgefeatB = _sc_gather(edge_raw_features,
                           jnp.concatenate([eidx1[H:].reshape(-1),
                                            eidx2.reshape(-1)]))

    Wkv0 = jnp.concatenate([Wk[0], Wv[0]], axis=1)
    Wkv1 = jnp.concatenate([Wk[1], Wv[1]], axis=1)

    d1_pk = (t1[:, None] - ts1).reshape(-1, K_NGH, 128)
    m1_pk = (ngh1 == 0).astype(jnp.float32).reshape(-1, K_NGH, 128)

    w0 = (Wq[0], Wkv0, Wo[0], merge_W1[0], merge_b1[0].reshape(1, -1),
          merge_W2[0], merge_b2[0].reshape(1, -1), freq, phase)
    out1a = _attn_layer(H, (nodefeatA, HF), (nodefeatA, HF), (nodefeatA, 0),
                        (edgefeatA, 0), d1_pk, m1_pk, *w0)
    out1b = _attn_layer(H, (nodefeatB, HF), (nodefeatB, HF), (nodefeatB, 0),
                        (edgefeatB, 0), d1_pk, m1_pk, *w0,
                        pk_off=HF // _FB)
    out1 = jnp.concatenate([out1a, out1b])

    d2_pk = (cut_time_l[:, None] - ts2).reshape(-1, K_NGH, 128)
    m2_pk = (ngh2 == 0).astype(jnp.float32).reshape(-1, K_NGH, 128)

    out2 = _attn_layer(B2, (out1, NGH2F), (nodefeatB, HF + H - B2),
                       (out1, 0), (edgefeatB, HF), d2_pk, m2_pk,
                       Wq[1], Wkv1, Wo[1], merge_W1[1],
                       merge_b1[1].reshape(1, -1), merge_W2[1],
                       merge_b2[1].reshape(1, -1), freq, phase)
    return out2
